# baseline (speedup 1.0000x reference)
import functools

import numpy as np
import jax
import jax.numpy as jnp
from jax import lax
from jax.experimental import pallas as pl
from jax.experimental.pallas import tpu as pltpu
from jax.experimental.pallas import tpu_sc as plsc

NC = 2
NS = 16
L = 16
NW = NC * NS

D = 768
NJ = D // L
CHUNK = 64
GRP = 16


def _rtne_bf16(xv):
    u = lax.bitcast_convert_type(xv, jnp.uint32)
    t = lax.shift_right_logical(u, jnp.uint32(16)) & jnp.uint32(1)
    r = (u + (t + jnp.uint32(0x7FFF))) & jnp.uint32(0xFFFF0000)
    return lax.bitcast_convert_type(r, jnp.float32)


def _sc_gate(x_hbm, w_hbm, o0_hbm, o1_hbm,
             buf0, buf1, w2v, wv, o0v, o1v, sem0, sem1, semw):
    n_tok = o0v.shape[0]
    n_chunks = n_tok // CHUNK
    wid = lax.axis_index("s") * NC + lax.axis_index("c")
    base = wid * n_tok

    pltpu.async_copy(w_hbm, w2v, semw).wait()
    for j in range(NJ):
        w0 = _rtne_bf16(w2v[0, pl.ds(j * L, L)])
        w1 = _rtne_bf16(w2v[1, pl.ds(j * L, L)])
        wv[pl.ds(j * L, L)] = w0 - w1
    lane = lax.iota(jnp.int32, L)
    bfly = [(lane ^ dist)[:, None] for dist in (8, 4, 2, 1)]
    gdn = lax.GatherDimensionNumbers(
        offset_dims=(), collapsed_slice_dims=(0,), start_index_map=(0,))

    def lane_sum(v):
        for idx in bfly:
            v = v + lax.gather(
                v, idx, gdn, (1,),
                mode=lax.GatherScatterMode.PROMISE_IN_BOUNDS)
        return v

    cp0 = pltpu.async_copy(x_hbm.at[pl.ds(base, CHUNK), :], buf0, sem0)
    cp1 = pltpu.async_copy(x_hbm.at[pl.ds(base + CHUNK, CHUNK), :], buf1, sem1)

    def do_chunk(c, buf):
        def group_body(g, _):
            t0 = c * CHUNK + g * GRP
            accs = [jnp.zeros((L,), jnp.float32) for _ in range(GRP)]
            for j in range(NJ):
                wj = wv[pl.ds(j * L, L)]
                for k in range(GRP):
                    t = g * GRP + k
                    xr = _rtne_bf16(buf[t, pl.ds(j * L, L)])
                    accs[k] = accs[k] + xr * wj
            res = jnp.zeros((L,), jnp.float32)
            for k in range(GRP):
                dv = lane_sum(accs[k])
                res = jnp.where((lane == k) & (dv >= 0.0), 1.0, res)
            o0v[pl.ds(t0, L)] = res
            o1v[pl.ds(t0, L)] = 1.0 - res
            return 0
        lax.fori_loop(0, CHUNK // GRP, group_body, 0)

    def pair_body(i, _):
        c0 = 2 * i
        cp0.wait()
        do_chunk(c0, buf0)

        @pl.when(c0 + 2 < n_chunks)
        def _():
            pltpu.async_copy(
                x_hbm.at[pl.ds(base + (c0 + 2) * CHUNK, CHUNK), :], buf0, sem0)

        cp1.wait()
        do_chunk(c0 + 1, buf1)

        @pl.when(c0 + 3 < n_chunks)
        def _():
            pltpu.async_copy(
                x_hbm.at[pl.ds(base + (c0 + 3) * CHUNK, CHUNK), :], buf1, sem1)
        return 0

    lax.fori_loop(0, n_chunks // 2, pair_body, 0)

    pltpu.sync_copy(o0v, o0_hbm.at[pl.ds(base, n_tok)])
    pltpu.sync_copy(o1v, o1_hbm.at[pl.ds(base, n_tok)])


def _sc_call(xr, W):
    n_rows = xr.shape[0]
    n_tok = n_rows // NW
    mesh = plsc.VectorSubcoreMesh(
        core_axis_name="c", subcore_axis_name="s",
        num_cores=NC, num_subcores=NS)
    f = functools.partial(
        pl.kernel, _sc_gate, mesh=mesh,
        out_type=[
            jax.ShapeDtypeStruct((n_rows,), jnp.float32),
            jax.ShapeDtypeStruct((n_rows,), jnp.float32),
        ],
        scratch_types=[
            pltpu.VMEM((CHUNK, D), jnp.float32),
            pltpu.VMEM((CHUNK, D), jnp.float32),
            pltpu.VMEM((2, D), jnp.float32),
            pltpu.VMEM((D,), jnp.float32),
            pltpu.VMEM((n_tok,), jnp.float32),
            pltpu.VMEM((n_tok,), jnp.float32),
            pltpu.SemaphoreType.DMA,
            pltpu.SemaphoreType.DMA,
            pltpu.SemaphoreType.DMA,
        ])()
    return f(xr, W)


def kernel(x, W):
    B, S, Dm = x.shape
    xr = x.reshape(B * S, Dm)
    o0, o1 = _sc_call(xr, W)
    return o0.reshape(B, S), o1.reshape(B, S)
